# baseline (speedup 1.0000x reference)
import functools

import jax
import jax.numpy as jnp
from jax.experimental import pallas as pl
from jax.experimental.pallas import tpu as pltpu

D_MODEL = 1024
POOL = 4096
K_MIN = 32
K_MAX = 256
TB = 128


def _proj_body(x_ref, wsel_ref, bsel_ref, ww_ref, bw_ref, w1_ref, b1_ref,
               w2_ref, b2_ref, sel_ref, w_ref, k_ref):
    x = x_ref[...]
    sel_ref[...] = jnp.dot(x, wsel_ref[...]) + bsel_ref[...]
    w_ref[...] = jnp.dot(x, ww_ref[...]) + bw_ref[...]
    h = jnp.maximum(jnp.dot(x, w1_ref[...]) + b1_ref[...], 0.0)
    c = jax.nn.sigmoid(jnp.dot(h, w2_ref[...].reshape(-1, 1))[:, 0] + b2_ref[0, 0])
    k_ref[...] = (K_MIN + c * (K_MAX - K_MIN)).astype(jnp.int32)[None, None, :]


def _run_proj(xf, W_sel, b_sel, W_w, b_w, W1, b1, W2, b2):
    n_tok = xf.shape[0]
    grid = (n_tok // TB,)
    const = lambda *_: (0, 0)
    out_shapes = (
        jax.ShapeDtypeStruct((n_tok, POOL), jnp.float32),
        jax.ShapeDtypeStruct((n_tok, POOL), jnp.float32),
        jax.ShapeDtypeStruct((n_tok // TB, 1, TB), jnp.int32),
    )
    return pl.pallas_call(
        _proj_body,
        grid=grid,
        in_specs=[
            pl.BlockSpec((TB, D_MODEL), lambda i: (i, 0)),
            pl.BlockSpec((D_MODEL, POOL), const),
            pl.BlockSpec((1, POOL), const),
            pl.BlockSpec((D_MODEL, POOL), const),
            pl.BlockSpec((1, POOL), const),
            pl.BlockSpec((D_MODEL, D_MODEL // 4), const),
            pl.BlockSpec((1, D_MODEL // 4), const),
            pl.BlockSpec((1, D_MODEL // 4), const),
            pl.BlockSpec((1, 1), const),
        ],
        out_specs=(
            pl.BlockSpec((TB, POOL), lambda i: (i, 0)),
            pl.BlockSpec((TB, POOL), lambda i: (i, 0)),
            pl.BlockSpec((1, 1, TB), lambda i: (i, 0, 0)),
        ),
        out_shape=out_shapes,
        compiler_params=pltpu.CompilerParams(
            dimension_semantics=("parallel",)),
    )(xf, W_sel, b_sel.reshape(1, POOL), W_w, b_w.reshape(1, POOL),
      W1, b1.reshape(1, -1), W2.reshape(1, -1), b2.reshape(1, 1))


def kernel(x, W_sel, b_sel, W_w, b_w, W1, b1, W2, b2):
    batch, seq, _ = x.shape
    xf = x.reshape(batch * seq, D_MODEL)
    sel, wsc, kv = _run_proj(xf, W_sel, b_sel, W_w, b_w, W1, b1, W2, b2)
    k_values = kv.reshape(batch, seq)
    selection_scores = sel.reshape(batch, seq, POOL)
    _, idx = jax.lax.top_k(selection_scores, K_MAX)
    pw = jnp.take_along_axis(wsc.reshape(batch, seq, POOL), idx, axis=-1)
    positions = jnp.broadcast_to(jnp.arange(K_MAX)[None, None, :],
                                 (batch, seq, K_MAX))
    mask = positions < k_values[..., None]
    pw = pw * mask.astype(jnp.float32)
    return (idx, pw, selection_scores, k_values)
